# baseline (speedup 1.0000x reference)
import functools

import jax
import jax.numpy as jnp
from jax import lax
from jax.experimental import pallas as pl
from jax.experimental.pallas import tpu as pltpu
from jax.experimental.pallas import tpu_sc as plsc

N = 16384
C = 1000
K = int(0.7 * N)
IGNORE = 0

B = 2048
NB = N // B

NSUB = 16
EPT = N // NSUB
NV = EPT // 16
NBIN = 256
NGRP = NBIN // 16


def _tc_body(xt_ref, tgt_ref, loss_ref):
    i = pl.program_id(0)
    x = xt_ref[...]
    tgt = tgt_ref[...]
    m = jnp.max(x, axis=0, keepdims=True)
    s = jnp.sum(jnp.exp(x - m), axis=0, keepdims=True)
    lse = m + jnp.log(s)
    rows = lax.broadcasted_iota(jnp.int32, (C, B), 0)
    picked = jnp.sum(jnp.where(rows == tgt, x, 0.0), axis=0, keepdims=True)
    loss_ref[...] = jnp.where(tgt == IGNORE, 0.0, lse - picked)[None]


def _sc_select_body(loss_hbm, out_hbm, elems_v, hist_v, mrg_v, merged_v,
                    obuf_v, shared):
    sid = lax.axis_index("s")
    cid = lax.axis_index("c")
    row = sid // 2
    col0 = (sid % 2) * EPT
    pltpu.sync_copy(loss_hbm.at[pl.ds(row, 1), pl.ds(col0, EPT)], elems_v)

    ones = jnp.ones((16,), jnp.float32)
    zeros = jnp.zeros((16,), jnp.float32)
    lane = lax.iota(jnp.int32, 16)

    remk = jnp.float32(K)
    cnt_gt = jnp.float32(0.0)
    sum_gt = jnp.float32(0.0)
    prefix = jnp.int32(0)

    for r in range(4):
        shift = 24 - 8 * r

        def zbody(k, _):
            hist_v[pl.ds(k * 16, 16)] = zeros
            return 0
        lax.fori_loop(0, 2 * NGRP, zbody, 0)

        def sbody(j, _, _shift=shift, _r=r, _prefix=prefix):
            x = elems_v[0, pl.ds(j * 16, 16)]
            bits = lax.bitcast_convert_type(x, jnp.int32)
            bucket = lax.shift_right_logical(bits, _shift) & 0xFF
            if _r == 0:
                okf = jnp.ones((16,), jnp.float32)
            else:
                ok = lax.shift_right_logical(bits, _shift + 8) == _prefix
                okf = ok.astype(jnp.float32)
            plsc.addupdate_scatter(hist_v, [bucket], okf)
            plsc.addupdate_scatter(hist_v, [bucket + NBIN], x * okf)
            return 0
        lax.fori_loop(0, NV, sbody, 0)

        pltpu.sync_copy(hist_v, shared.at[r, sid])
        plsc.subcore_barrier()
        pltpu.sync_copy(shared.at[r], mrg_v)

        def gbody(g, _):
            def tbody(t, acc):
                return acc + mrg_v[t, pl.ds(g * 16, 16)]
            merged_v[pl.ds(g * 16, 16)] = lax.fori_loop(0, NSUB, tbody, zeros)
            return 0
        lax.fori_loop(0, 2 * NGRP, gbody, 0)

        def fbody(i, carry):
            acc_c, acc_s, bstar, cnt_at, sum_at = carry
            g = NGRP - 1 - i
            h = merged_v[pl.ds(g * 16, 16)]
            hs = merged_v[pl.ds(NBIN + g * 16, 16)]
            rc = lax.rev(jnp.cumsum(lax.rev(h, (0,))), (0,))
            rcs = lax.rev(jnp.cumsum(lax.rev(hs, (0,))), (0,))
            gse = acc_c + rc - h
            gss = acc_s + rcs - hs
            m = jnp.logical_and(gse < remk, gse + h >= remk)
            mi = m.astype(jnp.int32)
            mf = m.astype(jnp.float32)
            bstar = bstar + jnp.sum(mi * (g * 16 + lane))
            cnt_at = cnt_at + jnp.sum(mf * gse)
            sum_at = sum_at + jnp.sum(mf * gss)
            return (acc_c + jnp.sum(h), acc_s + jnp.sum(hs),
                    bstar, cnt_at, sum_at)

        acc_c, acc_s, bstar, cnt_at, sum_at = lax.fori_loop(
            0, NGRP, fbody,
            (jnp.float32(0.0), jnp.float32(0.0), jnp.int32(0),
             jnp.float32(0.0), jnp.float32(0.0)))

        cnt_gt = cnt_gt + cnt_at
        sum_gt = sum_gt + sum_at
        remk = remk - cnt_at
        prefix = jnp.where(r == 0, bstar, (prefix << 8) | bstar)

    tval = jnp.max(lax.bitcast_convert_type(
        jnp.full((16,), prefix, jnp.int32), jnp.float32))
    mean = (sum_gt + remk * tval) * (1.0 / K)

    @pl.when(jnp.logical_and(sid == 0, cid == 0))
    def _write():
        obuf_v[...] = jnp.full((16,), mean, jnp.float32)
        pltpu.sync_copy(obuf_v, out_hbm)


@functools.cache
def _sc_select():
    mesh = plsc.VectorSubcoreMesh(core_axis_name="c", subcore_axis_name="s")
    return pl.kernel(
        _sc_select_body,
        mesh=mesh,
        compiler_params=pltpu.CompilerParams(needs_layout_passes=False),
        out_type=jax.ShapeDtypeStruct((16,), jnp.float32),
        scratch_types=[
            pltpu.VMEM((1, EPT), jnp.float32),
            pltpu.VMEM((2 * NBIN,), jnp.float32),
            pltpu.VMEM((NSUB, 2 * NBIN), jnp.float32),
            pltpu.VMEM((2 * NBIN,), jnp.float32),
            pltpu.VMEM((16,), jnp.float32),
            pltpu.VMEM_SHARED((4, NSUB, 2 * NBIN), jnp.float32),
        ],
    )


@jax.jit
def kernel(input, target):
    xt = input.T
    tgt2d = target.astype(jnp.int32)[None, :]

    loss = pl.pallas_call(
        _tc_body,
        grid=(NB,),
        in_specs=[
            pl.BlockSpec((C, B), lambda i: (0, i)),
            pl.BlockSpec((1, B), lambda i: (0, i)),
        ],
        out_specs=pl.BlockSpec((1, 1, B), lambda i: (i, 0, 0)),
        out_shape=jax.ShapeDtypeStruct((NB, 1, B), jnp.float32),
    )(xt, tgt2d)

    out = _sc_select()(loss.reshape(NB, B))
    return out[0]


# speedup vs baseline: 1.0429x; 1.0429x over previous
import functools

import jax
import jax.numpy as jnp
from jax import lax
from jax.experimental import pallas as pl
from jax.experimental.pallas import tpu as pltpu
from jax.experimental.pallas import tpu_sc as plsc

N = 16384
C = 1000
K = int(0.7 * N)
IGNORE = 0

B = 2048
NB = N // B

NSUB = 16
EPT = N // NSUB
NV = EPT // 16
NBIN = 256
NGRP = NBIN // 16


def _tc_body(xt_ref, tgt_ref, loss_ref):
    i = pl.program_id(0)
    x = xt_ref[...]
    tgt = tgt_ref[...]
    m = jnp.max(x, axis=0, keepdims=True)
    s = jnp.sum(jnp.exp(x - m), axis=0, keepdims=True)
    lse = m + jnp.log(s)
    rows = lax.broadcasted_iota(jnp.int32, (C, B), 0)
    picked = jnp.sum(jnp.where(rows == tgt, x, 0.0), axis=0, keepdims=True)
    loss_ref[...] = jnp.where(tgt == IGNORE, 0.0, lse - picked)[None]


def _sc_select_body(loss_hbm, out_hbm, elems_v, hist_v, mrg_v, merged_v,
                    obuf_v, shared):
    sid = lax.axis_index("s")
    cid = lax.axis_index("c")
    row = sid // 2
    col0 = (sid % 2) * EPT
    pltpu.sync_copy(loss_hbm.at[pl.ds(row, 1), pl.ds(col0, EPT)], elems_v)

    ones = jnp.ones((16,), jnp.float32)
    zeros = jnp.zeros((16,), jnp.float32)
    lane = lax.iota(jnp.int32, 16)

    remk = jnp.float32(K)
    cnt_gt = jnp.float32(0.0)
    sum_gt = jnp.float32(0.0)
    prefix = jnp.int32(0)

    for r in range(4):
        shift = 24 - 8 * r

        for k2 in range(2 * NGRP):
            hist_v[pl.ds(k2 * 16, 16)] = zeros

        for j in range(NV):
            x = elems_v[0, pl.ds(j * 16, 16)]
            bits = lax.bitcast_convert_type(x, jnp.int32)
            bucket = lax.shift_right_logical(bits, shift) & 0xFF
            if r == 0:
                okf = ones
            else:
                ok = lax.shift_right_logical(bits, shift + 8) == prefix
                okf = ok.astype(jnp.float32)
            plsc.addupdate_scatter(hist_v, [bucket], okf)
            plsc.addupdate_scatter(hist_v, [bucket + NBIN], x * okf)

        pltpu.sync_copy(hist_v, shared.at[r, sid])
        plsc.subcore_barrier()
        pltpu.sync_copy(shared.at[r], mrg_v)

        def gbody(g, _):
            acc = zeros
            for t in range(NSUB):
                acc = acc + mrg_v[t, pl.ds(g * 16, 16)]
            merged_v[pl.ds(g * 16, 16)] = acc
            return 0
        lax.fori_loop(0, 2 * NGRP, gbody, 0)

        sc_tot = [jnp.sum(merged_v[pl.ds(g * 16, 16)]) for g in range(NGRP)]
        ss_tot = [jnp.sum(merged_v[pl.ds(NBIN + g * 16, 16)])
                  for g in range(NGRP)]

        acc_c = jnp.float32(0.0)
        acc_s = jnp.float32(0.0)
        gidx = jnp.int32(0)
        above_c = jnp.float32(0.0)
        above_s = jnp.float32(0.0)
        for g in range(NGRP - 1, -1, -1):
            hit = jnp.logical_and(acc_c < remk, acc_c + sc_tot[g] >= remk)
            gidx = jnp.where(hit, jnp.int32(g), gidx)
            above_c = jnp.where(hit, acc_c, above_c)
            above_s = jnp.where(hit, acc_s, above_s)
            acc_c = acc_c + sc_tot[g]
            acc_s = acc_s + ss_tot[g]

        h = merged_v[pl.ds(gidx * 16, 16)]
        hs = merged_v[pl.ds(NBIN + gidx * 16, 16)]
        rc = lax.rev(jnp.cumsum(lax.rev(h, (0,))), (0,))
        rcs = lax.rev(jnp.cumsum(lax.rev(hs, (0,))), (0,))
        gse = above_c + rc - h
        gss = above_s + rcs - hs
        m = jnp.logical_and(gse < remk, gse + h >= remk)
        mi = m.astype(jnp.int32)
        mf = m.astype(jnp.float32)
        bstar = jnp.sum(mi * lane) + gidx * 16
        cnt_at = jnp.sum(mf * gse)
        sum_at = jnp.sum(mf * gss)

        cnt_gt = cnt_gt + cnt_at
        sum_gt = sum_gt + sum_at
        remk = remk - cnt_at
        prefix = jnp.where(r == 0, bstar, (prefix << 8) | bstar)

    tval = jnp.max(lax.bitcast_convert_type(
        jnp.full((16,), prefix, jnp.int32), jnp.float32))
    mean = (sum_gt + remk * tval) * (1.0 / K)

    @pl.when(jnp.logical_and(sid == 0, cid == 0))
    def _write():
        obuf_v[...] = jnp.full((16,), mean, jnp.float32)
        pltpu.sync_copy(obuf_v, out_hbm)


@functools.cache
def _sc_select():
    mesh = plsc.VectorSubcoreMesh(core_axis_name="c", subcore_axis_name="s")
    return pl.kernel(
        _sc_select_body,
        mesh=mesh,
        compiler_params=pltpu.CompilerParams(needs_layout_passes=False),
        out_type=jax.ShapeDtypeStruct((16,), jnp.float32),
        scratch_types=[
            pltpu.VMEM((1, EPT), jnp.float32),
            pltpu.VMEM((2 * NBIN,), jnp.float32),
            pltpu.VMEM((NSUB, 2 * NBIN), jnp.float32),
            pltpu.VMEM((2 * NBIN,), jnp.float32),
            pltpu.VMEM((16,), jnp.float32),
            pltpu.VMEM_SHARED((4, NSUB, 2 * NBIN), jnp.float32),
        ],
    )


@jax.jit
def kernel(input, target):
    xt = input.T
    tgt2d = target.astype(jnp.int32)[None, :]

    loss = pl.pallas_call(
        _tc_body,
        grid=(NB,),
        in_specs=[
            pl.BlockSpec((C, B), lambda i: (0, i)),
            pl.BlockSpec((1, B), lambda i: (0, i)),
        ],
        out_specs=pl.BlockSpec((1, 1, B), lambda i: (i, 0, 0)),
        out_shape=jax.ShapeDtypeStruct((NB, 1, B), jnp.float32),
    )(xt, tgt2d)

    out = _sc_select()(loss.reshape(NB, B))
    return out[0]


# speedup vs baseline: 1.9225x; 1.8435x over previous
import jax
import jax.numpy as jnp
from jax.experimental import pallas as pl
from jax.experimental.pallas import tpu as pltpu

N = 16384
C = 1000
K = int(0.7 * N)
IGNORE = 0

B = 2048
NB = N // B


def _body(xt_ref, tgt_ref, out_ref, loss_ref):
    i = pl.program_id(0)
    x = xt_ref[...]
    tgt = tgt_ref[...]
    m = jnp.max(x, axis=0, keepdims=True)
    s = jnp.sum(jnp.exp(x - m), axis=0, keepdims=True)
    lse = m + jnp.log(s)
    rows = jax.lax.broadcasted_iota(jnp.int32, (C, B), 0)
    picked = jnp.sum(jnp.where(rows == tgt, x, 0.0), axis=0, keepdims=True)
    loss_ref[pl.ds(i, 1), :] = jnp.where(tgt == IGNORE, 0.0, lse - picked)

    @pl.when(i == NB - 1)
    def _select():
        lx = loss_ref[...]
        bits = jax.lax.bitcast_convert_type(lx, jnp.int32)

        def srch(_, carry):
            lo, hi = carry
            mid = lo + (hi - lo) // 2
            cnt = jnp.sum(jnp.where(bits >= mid, 1, 0))
            return (jnp.where(cnt >= K, mid, lo),
                    jnp.where(cnt >= K, hi, mid))

        t, _ = jax.lax.fori_loop(
            0, 31, srch, (jnp.int32(0), jnp.int32(0x7F800001)))
        gt = bits > t
        cnt_gt = jnp.sum(jnp.where(gt, 1.0, 0.0))
        sum_gt = jnp.sum(jnp.where(gt, lx, 0.0))
        tv = jnp.max(jax.lax.bitcast_convert_type(
            jnp.full((8, 128), t, jnp.int32), jnp.float32))
        out_ref[0, 0] = (sum_gt + (jnp.float32(K) - cnt_gt) * tv) * (1.0 / K)


@jax.jit
def kernel(input, target):
    xt = input.T
    tgt2d = target.astype(jnp.int32)[None, :]

    out = pl.pallas_call(
        _body,
        grid=(NB,),
        in_specs=[
            pl.BlockSpec((C, B), lambda i: (0, i)),
            pl.BlockSpec((1, B), lambda i: (0, i)),
        ],
        out_specs=pl.BlockSpec(memory_space=pltpu.SMEM),
        out_shape=jax.ShapeDtypeStruct((1, 1), jnp.float32),
        scratch_shapes=[pltpu.VMEM((NB, B), jnp.float32)],
    )(xt, tgt2d)
    return out[0, 0]


# speedup vs baseline: 1.9722x; 1.0259x over previous
import jax
import jax.numpy as jnp
from jax.experimental import pallas as pl
from jax.experimental.pallas import tpu as pltpu

N = 16384
C = 1000
K = int(0.7 * N)
IGNORE = 0

B = 2048
NB = N // B
CH = 8
NCH = C // CH


def _body(xt_ref, tgt_ref, out_ref, loss_ref):
    i = pl.program_id(0)
    tgt = tgt_ref[...]

    acc = xt_ref[pl.ds(0, CH), :]
    for j in range(1, NCH):
        acc = jnp.maximum(acc, xt_ref[pl.ds(CH * j, CH), :])
    m = jnp.max(acc, axis=0, keepdims=True)

    rows8 = jax.lax.broadcasted_iota(jnp.int32, (CH, B), 0)
    oh8 = (rows8 == tgt % CH).astype(jnp.float32)
    tgtc = tgt // CH
    s8 = jnp.zeros((CH, B), jnp.float32)
    sel8 = jnp.zeros((CH, B), jnp.float32)
    for j in range(NCH):
        ch = xt_ref[pl.ds(CH * j, CH), :]
        s8 = s8 + jnp.exp(ch - m)
        sel8 = sel8 + ch * (tgtc == j).astype(jnp.float32)
    s = jnp.sum(s8, axis=0, keepdims=True)
    picked = jnp.sum(sel8 * oh8, axis=0, keepdims=True)
    lse = m + jnp.log(s)
    loss_ref[pl.ds(i, 1), :] = jnp.where(tgt == IGNORE, 0.0, lse - picked)

    @pl.when(i == NB - 1)
    def _select():
        lx = loss_ref[...]
        bits = jax.lax.bitcast_convert_type(lx, jnp.int32)

        def srch(_, carry):
            lo, hi = carry
            mid = lo + (hi - lo) // 2
            cnt = jnp.sum(jnp.where(bits >= mid, 1, 0))
            return (jnp.where(cnt >= K, mid, lo),
                    jnp.where(cnt >= K, hi, mid))

        t, _ = jax.lax.fori_loop(
            0, 31, srch, (jnp.int32(0), jnp.int32(0x7F800001)))
        gt = bits > t
        cnt_gt = jnp.sum(jnp.where(gt, 1.0, 0.0))
        sum_gt = jnp.sum(jnp.where(gt, lx, 0.0))
        tv = jnp.max(jax.lax.bitcast_convert_type(
            jnp.full((8, 128), t, jnp.int32), jnp.float32))
        out_ref[0, 0] = (sum_gt + (jnp.float32(K) - cnt_gt) * tv) * (1.0 / K)


@jax.jit
def kernel(input, target):
    xt = input.T
    tgt2d = target.astype(jnp.int32)[None, :]

    out = pl.pallas_call(
        _body,
        grid=(NB,),
        in_specs=[
            pl.BlockSpec((C, B), lambda i: (0, i)),
            pl.BlockSpec((1, B), lambda i: (0, i)),
        ],
        out_specs=pl.BlockSpec(memory_space=pltpu.SMEM),
        out_shape=jax.ShapeDtypeStruct((1, 1), jnp.float32),
        scratch_shapes=[pltpu.VMEM((NB, B), jnp.float32)],
    )(xt, tgt2d)
    return out[0, 0]
